# baseline (speedup 1.0000x reference)
import functools

import jax
import jax.numpy as jnp
from jax import lax
from jax.experimental import pallas as pl
from jax.experimental.pallas import tpu as pltpu
from jax.experimental.pallas import tpu_sc as plsc

DIM = 768
NUM_EXPERTS = 8
INTER = 2048
TOP_K = 2
T = 16
LANES = 16
DCH = DIM // LANES


def _lane_perm(v, perm):
    return v.at[perm].get(mode="promise_in_bounds")


def _butterfly(v, op):
    lane = lax.iota(jnp.int32, LANES)
    for k in (8, 4, 2, 1):
        v = op(v, _lane_perm(v, lane ^ k))
    return v


def _sc_router_body(x_hbm, gate_hbm, out_hbm, xv, gv, cv):
    wid = lax.axis_index("s") * 2 + lax.axis_index("c")

    @pl.when(wid < T)
    def _():
        t = wid
        pltpu.sync_copy(x_hbm.at[t], xv)
        pltpu.sync_copy(gate_hbm, gv)
        accs = [jnp.zeros((LANES,), jnp.float32) for _ in range(NUM_EXPERTS)]
        for j in range(DCH):
            xj = xv[pl.ds(j * LANES, LANES)]
            for e in range(NUM_EXPERTS):
                accs[e] = accs[e] + xj * gv[e, pl.ds(j * LANES, LANES)]
        lane = lax.iota(jnp.int32, LANES)
        validf = jnp.where(lane < NUM_EXPERTS, 1.0, 0.0)
        s = jnp.zeros((LANES,), jnp.float32)
        for e in range(NUM_EXPERTS):
            s = jnp.where(lane == e, _butterfly(accs[e], jnp.add), s)
        m = _butterfly(s * validf + (validf - 1.0) * 3.0e38, jnp.maximum)
        p = jnp.exp((s - m) * validf - 30.0 * (1.0 - validf)) * validf
        p = p / _butterfly(p, jnp.add)
        m1 = _butterfly(p, jnp.maximum)
        i1 = _butterfly(jnp.where(p == m1, lane, NUM_EXPERTS), jnp.minimum)
        oh1 = jnp.where(lane == i1, 1.0, 0.0)
        keep = (1.0 - oh1) * validf
        p_rest = p * keep - (1.0 - keep)
        m2 = _butterfly(p_rest, jnp.maximum)
        i2 = _butterfly(jnp.where(p_rest == m2, lane, NUM_EXPERTS),
                        jnp.minimum)
        oh2 = jnp.where(lane == i2, 1.0, 0.0)
        c = p * (oh1 + oh2)
        c = c / _butterfly(c, jnp.add)
        cv[...] = c
        pltpu.sync_copy(cv, out_hbm.at[t])


def _sc_router(x, gate_w):
    mesh = plsc.VectorSubcoreMesh(core_axis_name="c", subcore_axis_name="s")
    fn = functools.partial(
        pl.kernel,
        mesh=mesh,
        out_type=jax.ShapeDtypeStruct((T, LANES), jnp.float32),
        scratch_types=[
            pltpu.VMEM((DIM,), jnp.float32),
            pltpu.VMEM((NUM_EXPERTS, DIM), jnp.float32),
            pltpu.VMEM((LANES,), jnp.float32),
        ],
    )(_sc_router_body)
    return fn(x, gate_w)


def _moe_body(x_ref, c_ref, w1_ref, w2_ref, w3_ref, out_ref):
    e = pl.program_id(0)

    @pl.when(e == 0)
    def _init():
        out_ref[...] = jnp.zeros_like(out_ref)

    xv = x_ref[...]
    w1e = w1_ref[0]
    w3e = w3_ref[0]
    w2e = w2_ref[0]
    dn = (((1,), (1,)), ((), ()))
    h1 = lax.dot_general(xv, w1e, dn, preferred_element_type=jnp.float32)
    h3 = lax.dot_general(xv, w3e, dn, preferred_element_type=jnp.float32)
    h = h1 * lax.logistic(h1) * h3
    oute = lax.dot_general(h, w2e, dn, preferred_element_type=jnp.float32)
    eidx = lax.broadcasted_iota(jnp.int32, (T, LANES), 1)
    col = jnp.sum(jnp.where(eidx == e, c_ref[...], 0.0),
                  axis=-1, keepdims=True)
    out_ref[...] += col * oute


def kernel(x, gate_w, w1, w2, w3):
    original_shape = x.shape
    xf = x.reshape(-1, DIM)
    combine = _sc_router(xf, gate_w)
    out = pl.pallas_call(
        _moe_body,
        grid=(NUM_EXPERTS,),
        in_specs=[
            pl.BlockSpec((T, DIM), lambda e: (0, 0)),
            pl.BlockSpec((T, LANES), lambda e: (0, 0)),
            pl.BlockSpec((1, INTER, DIM), lambda e: (e, 0, 0)),
            pl.BlockSpec((1, DIM, INTER), lambda e: (e, 0, 0)),
            pl.BlockSpec((1, INTER, DIM), lambda e: (e, 0, 0)),
        ],
        out_specs=pl.BlockSpec((T, DIM), lambda e: (0, 0)),
        out_shape=jax.ShapeDtypeStruct((T, DIM), jnp.float32),
    )(xf, combine, w1, w2, w3)
    return out.reshape(original_shape)


# speedup vs baseline: 1.3908x; 1.3908x over previous
import jax
import jax.numpy as jnp
from jax import lax
from jax.experimental import pallas as pl
from jax.experimental.pallas import tpu as pltpu

DIM = 768
NUM_EXPERTS = 8
INTER = 2048
HALF = INTER // 2
TOP_K = 2
T = 16


def _routing_combine(x, gate_w):
    scores = lax.dot_general(
        x, gate_w, (((1,), (1,)), ((), ())),
        preferred_element_type=jnp.float32)
    m = jnp.max(scores, axis=-1, keepdims=True)
    p = jnp.exp(scores - m)
    p = p / jnp.sum(p, axis=-1, keepdims=True)
    eidx = lax.broadcasted_iota(jnp.int32, (T, NUM_EXPERTS), 1)
    m1 = jnp.max(p, axis=-1, keepdims=True)
    i1 = jnp.min(jnp.where(p == m1, eidx, NUM_EXPERTS), axis=-1, keepdims=True)
    oh1 = eidx == i1
    p_rest = jnp.where(oh1, -1.0, p)
    m2 = jnp.max(p_rest, axis=-1, keepdims=True)
    i2 = jnp.min(jnp.where(p_rest == m2, eidx, NUM_EXPERTS),
                 axis=-1, keepdims=True)
    oh2 = eidx == i2
    c = jnp.where(oh1 | oh2, p, 0.0)
    return c / jnp.sum(c, axis=-1, keepdims=True)


def _moe_body(x_ref, gate_ref, w1a_ref, w1b_ref, w3a_ref, w3b_ref,
              w2a_ref, w2b_ref, out_ref, c_ref):
    e = pl.program_id(0)

    @pl.when(e == 0)
    def _init():
        c_ref[...] = _routing_combine(x_ref[...], gate_ref[...])
        out_ref[...] = jnp.zeros_like(out_ref)

    xv = x_ref[...]
    dn = (((1,), (1,)), ((), ()))

    def halfblock(w1h, w3h):
        h1 = lax.dot_general(xv, w1h, dn, preferred_element_type=jnp.float32)
        h3 = lax.dot_general(xv, w3h, dn, preferred_element_type=jnp.float32)
        return h1 * lax.logistic(h1) * h3

    ha = halfblock(w1a_ref[0, 0], w3a_ref[0, 0])
    hb = halfblock(w1b_ref[0, 0], w3b_ref[0, 0])
    oute = (lax.dot_general(ha, w2a_ref[0], dn,
                            preferred_element_type=jnp.float32)
            + lax.dot_general(hb, w2b_ref[0], dn,
                              preferred_element_type=jnp.float32))
    eidx = lax.broadcasted_iota(jnp.int32, (T, NUM_EXPERTS), 1)
    col = jnp.sum(jnp.where(eidx == e, c_ref[...], 0.0),
                  axis=-1, keepdims=True)
    out_ref[...] += col * oute


def kernel(x, gate_w, w1, w2, w3):
    original_shape = x.shape
    xf = x.reshape(-1, DIM)
    w1r = w1.reshape(NUM_EXPERTS, 2, HALF, DIM)
    w3r = w3.reshape(NUM_EXPERTS, 2, HALF, DIM)
    out = pl.pallas_call(
        _moe_body,
        grid=(NUM_EXPERTS,),
        in_specs=[
            pl.BlockSpec((T, DIM), lambda e: (0, 0)),
            pl.BlockSpec((NUM_EXPERTS, DIM), lambda e: (0, 0)),
            pl.BlockSpec((1, 1, HALF, DIM), lambda e: (e, 0, 0, 0)),
            pl.BlockSpec((1, 1, HALF, DIM), lambda e: (e, 1, 0, 0)),
            pl.BlockSpec((1, 1, HALF, DIM), lambda e: (e, 0, 0, 0)),
            pl.BlockSpec((1, 1, HALF, DIM), lambda e: (e, 1, 0, 0)),
            pl.BlockSpec((1, DIM, HALF), lambda e: (e, 0, 0)),
            pl.BlockSpec((1, DIM, HALF), lambda e: (e, 0, 1)),
        ],
        out_specs=pl.BlockSpec((T, DIM), lambda e: (0, 0)),
        out_shape=jax.ShapeDtypeStruct((T, DIM), jnp.float32),
        scratch_shapes=[pltpu.VMEM((T, NUM_EXPERTS), jnp.float32)],
    )(xf, gate_w, w1r, w1r, w3r, w3r, w2, w2)
    return out.reshape(original_shape)
